# Initial kernel scaffold; baseline (speedup 1.0000x reference)
import jax
import jax.numpy as jnp
from jax.experimental import pallas as pl


def kernel(x, gate_W, gate_b, W1, b1, W2, b2, balance_weight):
    raise NotImplementedError("write your pallas kernel here")


# speedup vs baseline: 12.7404x; 12.7404x over previous
import functools

import jax
import jax.numpy as jnp
from jax import lax
from jax.experimental import pallas as pl
from jax.experimental.pallas import tpu as pltpu
from jax.experimental.pallas import tpu_sc as plsc

_TOKENS = 8192
_E = 64
_CAP = 153
_CPAD = 160
_NBLK = _E + 1
_ROWS = _NBLK * _CPAD
_TRASH = _E * _CPAD
_D = 768
_TBLK = 1024
_NTB = _TOKENS // _TBLK
_CHUNK = 128


def _route_body(x_ref, gw_ref, gb_ref, slot_ref, stats_ref, cnt_ref):
    i = pl.program_id(0)

    @pl.when(i == 0)
    def _():
        cnt_ref[...] = jnp.zeros_like(cnt_ref)
        stats_ref[...] = jnp.zeros_like(stats_ref)

    logits = jnp.dot(x_ref[...], gw_ref[...],
                     preferred_element_type=jnp.float32) + gb_ref[...]
    m = jnp.max(logits, axis=1, keepdims=True)
    eio = lax.broadcasted_iota(jnp.int32, (_TBLK, _E), 1)
    assign = jnp.min(jnp.where(logits == m, eio, _E), axis=1)
    oh = (eio == assign[:, None]).astype(jnp.float32)
    r_io = lax.broadcasted_iota(jnp.int32, (_TBLK, _TBLK), 0)
    c_io = lax.broadcasted_iota(jnp.int32, (_TBLK, _TBLK), 1)
    tril = (r_io >= c_io).astype(jnp.float32)
    cum = jnp.dot(tril, oh, preferred_element_type=jnp.float32)
    carry = cnt_ref[...]
    rank = jnp.sum(oh * (cum - 1.0 + carry), axis=1).astype(jnp.int32)
    slot_ref[0, 0, :] = jnp.where(rank < _CAP, assign * _CPAD + rank, _TRASH)
    new_cnt = carry + jnp.sum(oh, axis=0, keepdims=True)
    cnt_ref[...] = new_cnt

    @pl.when(i == _NTB - 1)
    def _():
        avg = jnp.mean(new_cnt)
        loss = jnp.sum((new_cnt - avg) ** 2) / _E
        stats_ref[0:1, 0:_E] = new_cnt
        stats_ref[1:2, 0:1] = loss.reshape(1, 1)


def _route(x, gate_W, gate_b):
    slot3, stats = pl.pallas_call(
        _route_body,
        grid=(_NTB,),
        in_specs=[
            pl.BlockSpec((_TBLK, _D), lambda i: (i, 0)),
            pl.BlockSpec((_D, _E), lambda i: (0, 0)),
            pl.BlockSpec((1, _E), lambda i: (0, 0)),
        ],
        out_specs=[
            pl.BlockSpec((1, 1, _TBLK), lambda i: (i, 0, 0)),
            pl.BlockSpec((8, 128), lambda i: (0, 0)),
        ],
        out_shape=[
            jax.ShapeDtypeStruct((_NTB, 1, _TBLK), jnp.int32),
            jax.ShapeDtypeStruct((8, 128), jnp.float32),
        ],
        scratch_shapes=[pltpu.VMEM((1, _E), jnp.float32)],
    )(x, gate_W, gate_b.reshape(1, _E))
    return slot3.reshape(_TOKENS), stats


def _expert_body(cnt_ref, x_ref, w1_ref, b1_ref, w2_ref, b2_ref, o_ref):
    cnt = cnt_ref[pl.program_id(0)]
    h = jnp.maximum(
        jnp.dot(x_ref[...], w1_ref[0], preferred_element_type=jnp.float32)
        + b1_ref[0], 0.0)
    o = jnp.dot(h, w2_ref[0], preferred_element_type=jnp.float32) + b2_ref[0]
    rows = lax.broadcasted_iota(jnp.int32, (_CPAD, 1), 0)
    o_ref[...] = jnp.where(rows < cnt, o, 0.0)


def _experts(xbuf, W1, b1, W2, b2, counts):
    wmap = lambda e: (jnp.minimum(e, _E - 1), 0, 0)
    return pl.pallas_call(
        _expert_body,
        grid=(_NBLK,),
        in_specs=[
            pl.BlockSpec(memory_space=pltpu.SMEM),
            pl.BlockSpec((_CPAD, _D), lambda e: (e, 0)),
            pl.BlockSpec((1, _D, _D), wmap),
            pl.BlockSpec((1, 1, _D), wmap),
            pl.BlockSpec((1, _D, _D), wmap),
            pl.BlockSpec((1, 1, _D), wmap),
        ],
        out_specs=pl.BlockSpec((_CPAD, _D), lambda e: (e, 0)),
        out_shape=jax.ShapeDtypeStruct((_ROWS, _D), jnp.float32),
    )(counts, xbuf, W1, b1.reshape(_E, 1, _D), W2, b2.reshape(_E, 1, _D))


def _sc_dispatch(x, slot):
    info = plsc.get_sparse_core_info()
    nw = info.num_cores * info.num_subcores
    per_w = _TOKENS // nw
    mesh = plsc.VectorSubcoreMesh(core_axis_name="c", subcore_axis_name="s")

    @functools.partial(
        pl.kernel,
        out_type=jax.ShapeDtypeStruct((_ROWS, _D), jnp.float32),
        mesh=mesh,
        scratch_types=[
            pltpu.VMEM((_CHUNK,), jnp.int32),
            pltpu.VMEM((_CHUNK, _D), jnp.float32),
            pltpu.SemaphoreType.DMA,
        ],
    )
    def k(x_hbm, slot_hbm, xbuf_hbm, idx_v, rows_v, sem):
        wid = lax.axis_index("s") * info.num_cores + lax.axis_index("c")
        for j in range(per_w // _CHUNK):
            base = wid * per_w + j * _CHUNK
            pltpu.sync_copy(slot_hbm.at[pl.ds(base, _CHUNK)], idx_v)
            pltpu.sync_copy(x_hbm.at[pl.ds(base, _CHUNK)], rows_v)
            pltpu.async_copy(rows_v, xbuf_hbm.at[idx_v], sem).wait()

    return k(x, slot)


def _sc_combine(obuf, slot):
    info = plsc.get_sparse_core_info()
    nw = info.num_cores * info.num_subcores
    per_w = _TOKENS // nw
    mesh = plsc.VectorSubcoreMesh(core_axis_name="c", subcore_axis_name="s")

    @functools.partial(
        pl.kernel,
        out_type=jax.ShapeDtypeStruct((_TOKENS, _D), jnp.float32),
        mesh=mesh,
        scratch_types=[
            pltpu.VMEM((_CHUNK,), jnp.int32),
            pltpu.VMEM((_CHUNK, _D), jnp.float32),
            pltpu.SemaphoreType.DMA,
        ],
    )
    def k(obuf_hbm, slot_hbm, out_hbm, idx_v, rows_v, sem):
        wid = lax.axis_index("s") * info.num_cores + lax.axis_index("c")
        for j in range(per_w // _CHUNK):
            base = wid * per_w + j * _CHUNK
            pltpu.sync_copy(slot_hbm.at[pl.ds(base, _CHUNK)], idx_v)
            pltpu.async_copy(obuf_hbm.at[idx_v], rows_v, sem).wait()
            pltpu.sync_copy(rows_v, out_hbm.at[pl.ds(base, _CHUNK)])

    return k(obuf, slot)


def kernel(x, gate_W, gate_b, W1, b1, W2, b2, balance_weight):
    slot, stats = _route(x, gate_W, gate_b)
    loss = stats[1, 0] * balance_weight
    counts = jnp.minimum(stats[0, :_E].astype(jnp.int32), _CAP)
    counts = jnp.concatenate([counts, jnp.zeros((1,), jnp.int32)])
    xbuf = _sc_dispatch(x, slot)
    obuf = _experts(xbuf, W1, b1, W2, b2, counts)
    out = _sc_combine(obuf, slot)
    return (out, loss)
